# Initial kernel scaffold; baseline (speedup 1.0000x reference)
import jax
import jax.numpy as jnp
from jax.experimental import pallas as pl


def kernel(inputs, table):
    raise NotImplementedError("write your pallas kernel here")


# speedup vs baseline: 4.5642x; 4.5642x over previous
import functools

import jax
import jax.numpy as jnp
from jax import lax
from jax.experimental import pallas as pl
from jax.experimental.pallas import tpu as pltpu
from jax.experimental.pallas import tpu_sc as plsc

BATCH = 4096
HIST = 50
EMBED_DIM = 64

_N = BATCH * HIST
_NC, _NS = 2, 16
_NW = _NC * _NS
_NPW = _N // _NW
_G = 128
_CH = 5
_CHN = _CH * _G
_NCHUNK = _NPW // _CHN


def _sc_gather(idx_hbm, table_hbm, out_hbm, idx_v, rows_v, sem):
    wid = lax.axis_index("s") * _NC + lax.axis_index("c")
    base = wid * _NPW
    pltpu.sync_copy(idx_hbm.at[pl.ds(base, _NPW)], idx_v)

    def body(j, _):
        copies = [
            pltpu.async_copy(
                table_hbm.at[idx_v.at[pl.ds(j * _CHN + b * _G, _G)]],
                rows_v.at[pl.ds(b * _G, _G)],
                sem,
            )
            for b in range(_CH)
        ]
        for c in copies:
            c.wait()
        pltpu.sync_copy(rows_v, out_hbm.at[pl.ds(base + j * _CHN, _CHN)])
        return ()

    lax.fori_loop(0, _NCHUNK, body, ())


_call = functools.partial(
    pl.kernel,
    mesh=plsc.VectorSubcoreMesh(core_axis_name="c", subcore_axis_name="s"),
    compiler_params=pltpu.CompilerParams(use_tc_tiling_on_sc=False),
    out_type=jax.ShapeDtypeStruct((_N, EMBED_DIM), jnp.float32),
    scratch_types=[
        pltpu.VMEM((_NPW,), jnp.int32),
        pltpu.VMEM((_CHN, EMBED_DIM), jnp.float32),
        pltpu.SemaphoreType.DMA,
    ],
)(_sc_gather)


def kernel(inputs, table):
    idx = inputs.astype(jnp.int32).reshape(_N)
    out = _call(idx, table)
    return out.reshape(BATCH, HIST, EMBED_DIM)


# speedup vs baseline: 4.6392x; 1.0164x over previous
import functools

import jax
import jax.numpy as jnp
from jax import lax
from jax.experimental import pallas as pl
from jax.experimental.pallas import tpu as pltpu
from jax.experimental.pallas import tpu_sc as plsc

BATCH = 4096
HIST = 50
EMBED_DIM = 64

_N = BATCH * HIST
_NC, _NS = 2, 16
_NW = _NC * _NS
_NPW = _N // _NW
_G = 128
_CH = 5
_CHN = _CH * _G
_NCHUNK = _NPW // _CHN


def _sc_gather(idx_hbm, table_hbm, out_hbm, idx_v, rows0, rows1, gsem, osem):
    wid = lax.axis_index("s") * _NC + lax.axis_index("c")
    base = wid * _NPW
    pltpu.sync_copy(idx_hbm.at[pl.ds(base, _NPW)], idx_v)

    bufs = (rows0, rows1)

    def fire(j, buf):
        return [
            pltpu.async_copy(
                table_hbm.at[idx_v.at[pl.ds(j * _CHN + k * _G, _G)]],
                buf.at[pl.ds(k * _G, _G)],
                gsem,
            )
            for k in range(_CH)
        ]

    gathers = {0: fire(0, bufs[0])}
    outs = {}
    for j in range(_NCHUNK):
        b = j % 2
        if j >= 1:
            outs.pop(j - 1).wait()
        if j + 1 < _NCHUNK:
            gathers[j + 1] = fire(j + 1, bufs[1 - b])
        for c in gathers.pop(j):
            c.wait()
        outs[j] = pltpu.async_copy(
            bufs[b], out_hbm.at[pl.ds(base + j * _CHN, _CHN)], osem
        )
    outs.pop(_NCHUNK - 1).wait()


_call = functools.partial(
    pl.kernel,
    mesh=plsc.VectorSubcoreMesh(core_axis_name="c", subcore_axis_name="s"),
    compiler_params=pltpu.CompilerParams(use_tc_tiling_on_sc=False),
    out_type=jax.ShapeDtypeStruct((_N, EMBED_DIM), jnp.float32),
    scratch_types=[
        pltpu.VMEM((_NPW,), jnp.int32),
        pltpu.VMEM((_CHN, EMBED_DIM), jnp.float32),
        pltpu.VMEM((_CHN, EMBED_DIM), jnp.float32),
        pltpu.SemaphoreType.DMA,
        pltpu.SemaphoreType.DMA,
    ],
)(_sc_gather)


def kernel(inputs, table):
    idx = inputs.astype(jnp.int32).reshape(_N)
    out = _call(idx, table)
    return out.reshape(BATCH, HIST, EMBED_DIM)
